# baseline (speedup 1.0000x reference)
import functools

import jax
import jax.numpy as jnp
from jax import lax
from jax.experimental import pallas as pl
from jax.experimental.pallas import tpu as pltpu
from jax.experimental.pallas import tpu_sc as plsc

N = 10000
E = 320000
D = 128
H = 128
C = 16

NUM_CORES = 2
NUM_SUB = 16
NUM_W = NUM_CORES * NUM_SUB
EW = E // NUM_W
N_PAD = 10240
ROWS_SUB = N_PAD // NUM_SUB
ZROWS = 160

KD = 2000
KS = 80
NCHUNK = EW // KS

TC_BLK = 1000


def _mesh():
    return plsc.VectorSubcoreMesh(
        core_axis_name="c", subcore_axis_name="s",
        num_cores=NUM_CORES, num_subcores=NUM_SUB)


def _deg_body(dst_hbm, out_hbm, idx_v, ones_v, zb_v, acc_sh, sem):
    c = lax.axis_index("c")
    s = lax.axis_index("s")
    base = (c * NUM_SUB + s) * EW

    @pl.loop(0, ROWS_SUB, step=16)
    def _fill(i):
        zb_v[pl.ds(i, 16)] = jnp.zeros((16,), jnp.float32)

    @pl.loop(0, KD, step=16)
    def _fill1(i):
        ones_v[pl.ds(i, 16)] = jnp.ones((16,), jnp.float32)

    pltpu.sync_copy(zb_v, acc_sh.at[pl.ds(s * ROWS_SUB, ROWS_SUB)])
    plsc.subcore_barrier()

    @pl.loop(0, EW, step=KD)
    def _edges(j):
        pltpu.async_copy(dst_hbm.at[pl.ds(base + j, KD)], idx_v, sem).wait()
        pltpu.sync_copy(ones_v, acc_sh.at[idx_v], add=True)

    plsc.subcore_barrier()
    pltpu.sync_copy(acc_sh.at[pl.ds(s * ROWS_SUB, ROWS_SUB)],
                    out_hbm.at[c, pl.ds(s * ROWS_SUB, ROWS_SUB)])


def _deg_call(dst):
    k = pl.kernel(
        _deg_body,
        out_type=jax.ShapeDtypeStruct((NUM_CORES, N_PAD), jnp.float32),
        mesh=_mesh(),
        scratch_types=[
            pltpu.VMEM((KD,), jnp.int32),
            pltpu.VMEM((KD,), jnp.float32),
            pltpu.VMEM((ROWS_SUB,), jnp.float32),
            pltpu.VMEM_SHARED((N_PAD,), jnp.float32),
            pltpu.SemaphoreType.DMA,
        ])
    return k(dst)


_UNROLL = 12
_NLOOP = (NCHUNK - 5) // _UNROLL * _UNROLL


def _scatter_body(eidx_hbm, g_hbm, out_hbm,
                  ib0, ib1, ib2, ib3, rb0, rb1, rb2, acc_sh,
                  si0, si1, si2, si3, sg0, sg1, sg2):
    c = lax.axis_index("c")
    s = lax.axis_index("s")
    w = c * NUM_SUB + s
    ibs = (ib0, ib1, ib2, ib3)
    sis = (si0, si1, si2, si3)
    rbs = (rb0, rb1, rb2)
    sgs = (sg0, sg1, sg2)

    @pl.loop(0, KS)
    def _zr(r):
        @pl.loop(0, H, step=16)
        def _zc(j):
            rb0[pl.ds(r, 1), pl.ds(j, 16)] = jnp.zeros((1, 16), jnp.float32)

    @pl.loop(0, ROWS_SUB, step=KS)
    def _zero(r):
        pltpu.sync_copy(rb0, acc_sh.at[pl.ds(s * ROWS_SUB + r, KS), :])

    plsc.subcore_barrier()

    pltpu.sync_copy(eidx_hbm.at[w, 0], ib0)
    pltpu.sync_copy(eidx_hbm.at[w, 1], ib1)
    pltpu.async_copy(eidx_hbm.at[w, 2], ib2, si2)
    pltpu.async_copy(g_hbm.at[ib0.at[0]], rb0, sg0)
    pltpu.async_copy(g_hbm.at[ib1.at[0]], rb1, sg1)

    def chunk_step(m, q3, q4, do_idx, do_gather):
        rb, sg = rbs[q3], sgs[q3]
        ib = ibs[q4]
        ib3n, si3n = ibs[(q4 + 3) % 4], sis[(q4 + 3) % 4]
        rb2n, sg2n = rbs[(q3 + 2) % 3], sgs[(q3 + 2) % 3]
        si2n = sis[(q4 + 2) % 4]

        pltpu.make_async_copy(g_hbm.at[ib.at[0]], rb, sg).wait()
        pltpu.sync_copy(rb, acc_sh.at[ib.at[1]], add=True)
        if do_idx:
            pltpu.async_copy(eidx_hbm.at[w, m + 3], ib3n, si3n)
        if do_gather:
            pltpu.make_async_copy(eidx_hbm.at[w, m + 2], ibs[(q4 + 2) % 4],
                                  si2n).wait()
            pltpu.async_copy(g_hbm.at[ibs[(q4 + 2) % 4].at[0]], rb2n, sg2n)

    @pl.loop(0, _NLOOP, step=_UNROLL)
    def _edges(j):
        for t in range(_UNROLL):
            chunk_step(j + t, t % 3, t % 4, True, True)

    for m in range(_NLOOP, NCHUNK):
        chunk_step(m, m % 3, m % 4, m + 3 < NCHUNK, m + 2 < NCHUNK)

    plsc.subcore_barrier()
    pltpu.sync_copy(acc_sh.at[pl.ds(s * ROWS_SUB, ROWS_SUB), :],
                    out_hbm.at[c, pl.ds(s * ROWS_SUB, ROWS_SUB), :])


def _scatter_call(eidx_packed, g):
    k = pl.kernel(
        _scatter_body,
        out_type=jax.ShapeDtypeStruct((NUM_CORES, N_PAD, H), jnp.float32),
        mesh=_mesh(),
        scratch_types=[
            pltpu.VMEM((2, KS), jnp.int32),
            pltpu.VMEM((2, KS), jnp.int32),
            pltpu.VMEM((2, KS), jnp.int32),
            pltpu.VMEM((2, KS), jnp.int32),
            pltpu.VMEM((KS, H), jnp.float32),
            pltpu.VMEM((KS, H), jnp.float32),
            pltpu.VMEM((KS, H), jnp.float32),
            pltpu.VMEM_SHARED((N_PAD, H), jnp.float32),
            pltpu.SemaphoreType.DMA,
            pltpu.SemaphoreType.DMA,
            pltpu.SemaphoreType.DMA,
            pltpu.SemaphoreType.DMA,
            pltpu.SemaphoreType.DMA,
            pltpu.SemaphoreType.DMA,
            pltpu.SemaphoreType.DMA,
        ])
    return k(eidx_packed, g)


def _mm1_body(x_ref, w_ref, h_ref):
    h_ref[...] = jnp.dot(x_ref[...], w_ref[...],
                         preferred_element_type=jnp.float32,
                         precision=lax.Precision.HIGHEST)


def _mm1_call(x, W1):
    grid = N // TC_BLK
    return pl.pallas_call(
        _mm1_body,
        grid=(grid,),
        in_specs=[
            pl.BlockSpec((TC_BLK, D), lambda i: (i, 0)),
            pl.BlockSpec((D, H), lambda i: (0, 0)),
        ],
        out_specs=pl.BlockSpec((TC_BLK, H), lambda i: (i, 0)),
        out_shape=jax.ShapeDtypeStruct((N, H), jnp.float32),
    )(x, W1)


def _tc1_body(h_ref, degp_ref, g_ref, dinv_ref):
    deg = degp_ref[:, 0] + degp_ref[:, 1] + 1.0
    dinv = lax.rsqrt(deg)[:, None]
    g_ref[...] = h_ref[...] * dinv
    dinv_ref[...] = dinv


def _tc1_call(h, degp):
    grid = N // TC_BLK
    degp_t = jnp.swapaxes(degp, 0, 1)
    return pl.pallas_call(
        _tc1_body,
        grid=(grid,),
        in_specs=[
            pl.BlockSpec((TC_BLK, H), lambda i: (i, 0)),
            pl.BlockSpec((TC_BLK, NUM_CORES), lambda i: (i, 0)),
        ],
        out_specs=[
            pl.BlockSpec((TC_BLK, H), lambda i: (i, 0)),
            pl.BlockSpec((TC_BLK, 1), lambda i: (i, 0)),
        ],
        out_shape=[
            jax.ShapeDtypeStruct((N, H), jnp.float32),
            jax.ShapeDtypeStruct((N, 1), jnp.float32),
        ],
    )(h, degp_t)


def _tc2_body(p_ref, g_ref, dinv_ref, b_ref, w_ref, g2_ref):
    acc = p_ref[0] + p_ref[1] + g_ref[...]
    dinv = dinv_ref[...]
    t = jnp.tanh(acc * dinv + b_ref[...][None, :])
    h2 = jnp.dot(t, w_ref[...],
                 preferred_element_type=jnp.float32,
                 precision=lax.Precision.HIGHEST)
    g2_ref[...] = h2 * dinv


def _tc2_call(p1, g1, dinv, b1, W2):
    grid = N // TC_BLK
    return pl.pallas_call(
        _tc2_body,
        grid=(grid,),
        in_specs=[
            pl.BlockSpec((NUM_CORES, TC_BLK, H), lambda i: (0, i, 0)),
            pl.BlockSpec((TC_BLK, H), lambda i: (i, 0)),
            pl.BlockSpec((TC_BLK, 1), lambda i: (i, 0)),
            pl.BlockSpec((H,), lambda i: (0,)),
            pl.BlockSpec((H, H), lambda i: (0, 0)),
        ],
        out_specs=pl.BlockSpec((TC_BLK, H), lambda i: (i, 0)),
        out_shape=jax.ShapeDtypeStruct((N, H), jnp.float32),
    )(p1, g1, dinv, b1, W2)


def _tc3_body(p_ref, g_ref, dinv_ref, b_ref, wl_ref, bl_ref, o_ref):
    acc = p_ref[0] + p_ref[1] + g_ref[...]
    dinv = dinv_ref[...]
    t = jnp.tanh(acc * dinv + b_ref[...][None, :])
    o = jnp.dot(t, wl_ref[...],
                preferred_element_type=jnp.float32,
                precision=lax.Precision.HIGHEST) + bl_ref[...][None, :]
    m = jnp.max(o, axis=1, keepdims=True)
    e = jnp.exp(o - m)
    lse = jnp.log(jnp.sum(e, axis=1, keepdims=True))
    o_ref[...] = o - m - lse


def _tc3_call(p2, g2, dinv, b2, Wlin, blin):
    grid = N // TC_BLK
    return pl.pallas_call(
        _tc3_body,
        grid=(grid,),
        in_specs=[
            pl.BlockSpec((NUM_CORES, TC_BLK, H), lambda i: (0, i, 0)),
            pl.BlockSpec((TC_BLK, H), lambda i: (i, 0)),
            pl.BlockSpec((TC_BLK, 1), lambda i: (i, 0)),
            pl.BlockSpec((H,), lambda i: (0,)),
            pl.BlockSpec((H, C), lambda i: (0, 0)),
            pl.BlockSpec((C,), lambda i: (0,)),
        ],
        out_specs=pl.BlockSpec((TC_BLK, C), lambda i: (i, 0)),
        out_shape=jax.ShapeDtypeStruct((N, C), jnp.float32),
    )(p2, g2, dinv, b2, Wlin, blin)


def kernel(x, edge_index, W1, b1, W2, b2, Wlin, blin):
    dst = edge_index[1]
    eidx_packed = jnp.transpose(
        edge_index.reshape(2, NUM_W, NCHUNK, KS), (1, 2, 0, 3))
    degp = _deg_call(dst)
    h1 = _mm1_call(x, W1)
    g1, dinv = _tc1_call(h1, degp)
    p1 = _scatter_call(eidx_packed, g1)
    g2 = _tc2_call(p1, g1, dinv, b1, W2)
    p2 = _scatter_call(eidx_packed, g2)
    return _tc3_call(p2, g2, dinv, b2, Wlin, blin)


# speedup vs baseline: 1.0804x; 1.0804x over previous
import functools

import jax
import jax.numpy as jnp
from jax import lax
from jax.experimental import pallas as pl
from jax.experimental.pallas import tpu as pltpu
from jax.experimental.pallas import tpu_sc as plsc

N = 10000
E = 320000
D = 128
H = 128
C = 16

NUM_CORES = 2
NUM_SUB = 16
NUM_W = NUM_CORES * NUM_SUB
EW = E // NUM_W
N_PAD = 10240
ROWS_SUB = N_PAD // NUM_SUB
ZROWS = 160

KD = 2000
KS = 80
NCHUNK = EW // KS

TC_BLK = 2000


def _mesh():
    return plsc.VectorSubcoreMesh(
        core_axis_name="c", subcore_axis_name="s",
        num_cores=NUM_CORES, num_subcores=NUM_SUB)


def _deg_body(dst_hbm, out_hbm, idx_v, ones_v, zb_v, acc_sh, sem):
    c = lax.axis_index("c")
    s = lax.axis_index("s")
    base = (c * NUM_SUB + s) * EW

    @pl.loop(0, ROWS_SUB, step=16)
    def _fill(i):
        zb_v[pl.ds(i, 16)] = jnp.zeros((16,), jnp.float32)

    @pl.loop(0, KD, step=16)
    def _fill1(i):
        ones_v[pl.ds(i, 16)] = jnp.ones((16,), jnp.float32)

    pltpu.sync_copy(zb_v, acc_sh.at[pl.ds(s * ROWS_SUB, ROWS_SUB)])
    plsc.subcore_barrier()

    @pl.loop(0, EW, step=KD)
    def _edges(j):
        pltpu.async_copy(dst_hbm.at[pl.ds(base + j, KD)], idx_v, sem).wait()
        pltpu.sync_copy(ones_v, acc_sh.at[idx_v], add=True)

    plsc.subcore_barrier()
    pltpu.sync_copy(acc_sh.at[pl.ds(s * ROWS_SUB, ROWS_SUB)],
                    out_hbm.at[c, pl.ds(s * ROWS_SUB, ROWS_SUB)])


def _deg_call(dst):
    k = pl.kernel(
        _deg_body,
        out_type=jax.ShapeDtypeStruct((NUM_CORES, N_PAD), jnp.float32),
        mesh=_mesh(),
        scratch_types=[
            pltpu.VMEM((KD,), jnp.int32),
            pltpu.VMEM((KD,), jnp.float32),
            pltpu.VMEM((ROWS_SUB,), jnp.float32),
            pltpu.VMEM_SHARED((N_PAD,), jnp.float32),
            pltpu.SemaphoreType.DMA,
        ])
    return k(dst)


_UNROLL = 12
_NLOOP = (NCHUNK - 5) // _UNROLL * _UNROLL


def _scatter_body(eidx_hbm, g_hbm, out_hbm,
                  ib0, ib1, ib2, ib3, rb0, rb1, rb2, acc_sh,
                  si0, si1, si2, si3, sg0, sg1, sg2):
    c = lax.axis_index("c")
    s = lax.axis_index("s")
    w = c * NUM_SUB + s
    ibs = (ib0, ib1, ib2, ib3)
    sis = (si0, si1, si2, si3)
    rbs = (rb0, rb1, rb2)
    sgs = (sg0, sg1, sg2)

    @pl.loop(0, KS)
    def _zr(r):
        @pl.loop(0, H, step=16)
        def _zc(j):
            rb2[pl.ds(r, 1), pl.ds(j, 16)] = jnp.zeros((1, 16), jnp.float32)

    pltpu.sync_copy(eidx_hbm.at[w, 0], ib0)
    pltpu.sync_copy(eidx_hbm.at[w, 1], ib1)
    pltpu.async_copy(eidx_hbm.at[w, 2], ib2, si2)
    pltpu.async_copy(g_hbm.at[ib0.at[0]], rb0, sg0)
    pltpu.async_copy(g_hbm.at[ib1.at[0]], rb1, sg1)

    @pl.loop(0, ROWS_SUB, step=KS)
    def _zero(r):
        pltpu.sync_copy(rb2, acc_sh.at[pl.ds(s * ROWS_SUB + r, KS), :])

    plsc.subcore_barrier()

    def chunk_step(m, q3, q4, do_idx, do_gather):
        rb, sg = rbs[q3], sgs[q3]
        ib = ibs[q4]
        ib3n, si3n = ibs[(q4 + 3) % 4], sis[(q4 + 3) % 4]
        rb2n, sg2n = rbs[(q3 + 2) % 3], sgs[(q3 + 2) % 3]
        si2n = sis[(q4 + 2) % 4]

        pltpu.make_async_copy(g_hbm.at[ib.at[0]], rb, sg).wait()
        pltpu.sync_copy(rb, acc_sh.at[ib.at[1]], add=True)
        if do_idx:
            pltpu.async_copy(eidx_hbm.at[w, m + 3], ib3n, si3n)
        if do_gather:
            pltpu.make_async_copy(eidx_hbm.at[w, m + 2], ibs[(q4 + 2) % 4],
                                  si2n).wait()
            pltpu.async_copy(g_hbm.at[ibs[(q4 + 2) % 4].at[0]], rb2n, sg2n)

    @pl.loop(0, _NLOOP, step=_UNROLL)
    def _edges(j):
        for t in range(_UNROLL):
            chunk_step(j + t, t % 3, t % 4, True, True)

    for m in range(_NLOOP, NCHUNK):
        chunk_step(m, m % 3, m % 4, m + 3 < NCHUNK, m + 2 < NCHUNK)

    plsc.subcore_barrier()
    pltpu.sync_copy(acc_sh.at[pl.ds(s * ROWS_SUB, ROWS_SUB), :],
                    out_hbm.at[c, pl.ds(s * ROWS_SUB, ROWS_SUB), :])


def _scatter_call(eidx_packed, g):
    k = pl.kernel(
        _scatter_body,
        out_type=jax.ShapeDtypeStruct((NUM_CORES, N_PAD, H), jnp.float32),
        mesh=_mesh(),
        scratch_types=[
            pltpu.VMEM((2, KS), jnp.int32),
            pltpu.VMEM((2, KS), jnp.int32),
            pltpu.VMEM((2, KS), jnp.int32),
            pltpu.VMEM((2, KS), jnp.int32),
            pltpu.VMEM((KS, H), jnp.float32),
            pltpu.VMEM((KS, H), jnp.float32),
            pltpu.VMEM((KS, H), jnp.float32),
            pltpu.VMEM_SHARED((N_PAD, H), jnp.float32),
            pltpu.SemaphoreType.DMA,
            pltpu.SemaphoreType.DMA,
            pltpu.SemaphoreType.DMA,
            pltpu.SemaphoreType.DMA,
            pltpu.SemaphoreType.DMA,
            pltpu.SemaphoreType.DMA,
            pltpu.SemaphoreType.DMA,
        ])
    return k(eidx_packed, g)


def _tc1_body(x_ref, w_ref, degp_ref, g_ref, dinv_ref):
    deg = degp_ref[:, 0] + degp_ref[:, 1] + 1.0
    dinv = lax.rsqrt(deg)[:, None]
    h = jnp.dot(x_ref[...], w_ref[...],
                preferred_element_type=jnp.float32,
                precision=lax.Precision.HIGHEST)
    g_ref[...] = h * dinv
    dinv_ref[...] = dinv


def _tc1_call(x, W1, degp):
    grid = N // TC_BLK
    degp_t = jnp.swapaxes(degp, 0, 1)
    return pl.pallas_call(
        _tc1_body,
        grid=(grid,),
        in_specs=[
            pl.BlockSpec((TC_BLK, D), lambda i: (i, 0)),
            pl.BlockSpec((D, H), lambda i: (0, 0)),
            pl.BlockSpec((TC_BLK, NUM_CORES), lambda i: (i, 0)),
        ],
        out_specs=[
            pl.BlockSpec((TC_BLK, H), lambda i: (i, 0)),
            pl.BlockSpec((TC_BLK, 1), lambda i: (i, 0)),
        ],
        out_shape=[
            jax.ShapeDtypeStruct((N, H), jnp.float32),
            jax.ShapeDtypeStruct((N, 1), jnp.float32),
        ],
    )(x, W1, degp_t)


def _tc2_body(p_ref, g_ref, dinv_ref, b_ref, w_ref, g2_ref):
    acc = p_ref[0] + p_ref[1] + g_ref[...]
    dinv = dinv_ref[...]
    t = jnp.tanh(acc * dinv + b_ref[...][None, :])
    h2 = jnp.dot(t, w_ref[...],
                 preferred_element_type=jnp.float32,
                 precision=lax.Precision.HIGHEST)
    g2_ref[...] = h2 * dinv


def _tc2_call(p1, g1, dinv, b1, W2):
    grid = N // TC_BLK
    return pl.pallas_call(
        _tc2_body,
        grid=(grid,),
        in_specs=[
            pl.BlockSpec((NUM_CORES, TC_BLK, H), lambda i: (0, i, 0)),
            pl.BlockSpec((TC_BLK, H), lambda i: (i, 0)),
            pl.BlockSpec((TC_BLK, 1), lambda i: (i, 0)),
            pl.BlockSpec((H,), lambda i: (0,)),
            pl.BlockSpec((H, H), lambda i: (0, 0)),
        ],
        out_specs=pl.BlockSpec((TC_BLK, H), lambda i: (i, 0)),
        out_shape=jax.ShapeDtypeStruct((N, H), jnp.float32),
    )(p1, g1, dinv, b1, W2)


def _tc3_body(p_ref, g_ref, dinv_ref, b_ref, wl_ref, bl_ref, o_ref):
    acc = p_ref[0] + p_ref[1] + g_ref[...]
    dinv = dinv_ref[...]
    t = jnp.tanh(acc * dinv + b_ref[...][None, :])
    o = jnp.dot(t, wl_ref[...],
                preferred_element_type=jnp.float32,
                precision=lax.Precision.HIGHEST) + bl_ref[...][None, :]
    m = jnp.max(o, axis=1, keepdims=True)
    e = jnp.exp(o - m)
    lse = jnp.log(jnp.sum(e, axis=1, keepdims=True))
    o_ref[...] = o - m - lse


def _tc3_call(p2, g2, dinv, b2, Wlin, blin):
    grid = N // TC_BLK
    return pl.pallas_call(
        _tc3_body,
        grid=(grid,),
        in_specs=[
            pl.BlockSpec((NUM_CORES, TC_BLK, H), lambda i: (0, i, 0)),
            pl.BlockSpec((TC_BLK, H), lambda i: (i, 0)),
            pl.BlockSpec((TC_BLK, 1), lambda i: (i, 0)),
            pl.BlockSpec((H,), lambda i: (0,)),
            pl.BlockSpec((H, C), lambda i: (0, 0)),
            pl.BlockSpec((C,), lambda i: (0,)),
        ],
        out_specs=pl.BlockSpec((TC_BLK, C), lambda i: (i, 0)),
        out_shape=jax.ShapeDtypeStruct((N, C), jnp.float32),
    )(p2, g2, dinv, b2, Wlin, blin)


def kernel(x, edge_index, W1, b1, W2, b2, Wlin, blin):
    dst = edge_index[1]
    eidx_packed = jnp.transpose(
        edge_index.reshape(2, NUM_W, NCHUNK, KS), (1, 2, 0, 3))
    degp = _deg_call(dst)
    g1, dinv = _tc1_call(x, W1, degp)
    p1 = _scatter_call(eidx_packed, g1)
    g2 = _tc2_call(p1, g1, dinv, b1, W2)
    p2 = _scatter_call(eidx_packed, g2)
    return _tc3_call(p2, g2, dinv, b2, Wlin, blin)


# speedup vs baseline: 1.0804x; 1.0000x over previous
import jax
import jax.numpy as jnp
from jax import lax
from jax.experimental import pallas as pl
from jax.experimental.pallas import tpu as pltpu
from jax.experimental.pallas import tpu_sc as plsc

N = 10000
E = 320000
D = 128
H = 128
C = 16

NUM_CORES = 2
NUM_SUB = 16
NUM_W = NUM_CORES * NUM_SUB
EW = E // NUM_W
N_PAD = 10240
ROWS_SUB = N_PAD // NUM_SUB

KD = 2000
KS = 80
NCHUNK = EW // KS

TC_BLK = 2000


def _mesh():
    return plsc.VectorSubcoreMesh(
        core_axis_name="c", subcore_axis_name="s",
        num_cores=NUM_CORES, num_subcores=NUM_SUB)


def _deg_body(dst_hbm, out_hbm, idx_v, ones_v, zb_v, acc_sh, sem):
    c = lax.axis_index("c")
    s = lax.axis_index("s")
    base = (c * NUM_SUB + s) * EW

    @pl.loop(0, ROWS_SUB, step=16)
    def _fill(i):
        zb_v[pl.ds(i, 16)] = jnp.zeros((16,), jnp.float32)

    @pl.loop(0, KD, step=16)
    def _fill1(i):
        ones_v[pl.ds(i, 16)] = jnp.ones((16,), jnp.float32)

    pltpu.sync_copy(zb_v, acc_sh.at[pl.ds(s * ROWS_SUB, ROWS_SUB)])
    plsc.subcore_barrier()

    @pl.loop(0, EW, step=KD)
    def _edges(j):
        pltpu.async_copy(dst_hbm.at[pl.ds(base + j, KD)], idx_v, sem).wait()
        pltpu.sync_copy(ones_v, acc_sh.at[idx_v], add=True)

    plsc.subcore_barrier()
    pltpu.sync_copy(acc_sh.at[pl.ds(s * ROWS_SUB, ROWS_SUB)],
                    out_hbm.at[c, pl.ds(s * ROWS_SUB, ROWS_SUB)])


def _deg_call(dst):
    k = pl.kernel(
        _deg_body,
        out_type=jax.ShapeDtypeStruct((NUM_CORES, N_PAD), jnp.float32),
        mesh=_mesh(),
        scratch_types=[
            pltpu.VMEM((KD,), jnp.int32),
            pltpu.VMEM((KD,), jnp.float32),
            pltpu.VMEM((ROWS_SUB,), jnp.float32),
            pltpu.VMEM_SHARED((N_PAD,), jnp.float32),
            pltpu.SemaphoreType.DMA,
        ])
    return k(dst)


_UNROLL = 12
_NLOOP = (NCHUNK - 5) // _UNROLL * _UNROLL


def _scatter_body(eidx_hbm, g_hbm, out_hbm,
                  ib0, ib1, ib2, ib3, rb0, rb1, rb2, acc_sh,
                  si0, si1, si2, si3, sg0, sg1, sg2):
    c = lax.axis_index("c")
    s = lax.axis_index("s")
    w = c * NUM_SUB + s
    ibs = (ib0, ib1, ib2, ib3)
    sis = (si0, si1, si2, si3)
    rbs = (rb0, rb1, rb2)
    sgs = (sg0, sg1, sg2)

    @pl.loop(0, KS)
    def _zr(r):
        @pl.loop(0, H, step=16)
        def _zc(j):
            rb2[pl.ds(r, 1), pl.ds(j, 16)] = jnp.zeros((1, 16), jnp.float32)

    pltpu.sync_copy(eidx_hbm.at[w, 0], ib0)
    pltpu.sync_copy(eidx_hbm.at[w, 1], ib1)
    pltpu.async_copy(eidx_hbm.at[w, 2], ib2, si2)
    pltpu.async_copy(g_hbm.at[ib0.at[0]], rb0, sg0)
    pltpu.async_copy(g_hbm.at[ib1.at[0]], rb1, sg1)

    @pl.loop(0, ROWS_SUB, step=KS)
    def _zero(r):
        pltpu.sync_copy(rb2, acc_sh.at[pl.ds(s * ROWS_SUB + r, KS), :])

    plsc.subcore_barrier()

    def chunk_step(m, q3, q4, do_idx, do_gather):
        rb, sg = rbs[q3], sgs[q3]
        ib = ibs[q4]
        ib3n, si3n = ibs[(q4 + 3) % 4], sis[(q4 + 3) % 4]
        rb2n, sg2n = rbs[(q3 + 2) % 3], sgs[(q3 + 2) % 3]
        si2n = sis[(q4 + 2) % 4]

        pltpu.make_async_copy(g_hbm.at[ib.at[0]], rb, sg).wait()
        pltpu.sync_copy(rb, acc_sh.at[ib.at[1]], add=True)
        if do_idx:
            pltpu.async_copy(eidx_hbm.at[w, m + 3], ib3n, si3n)
        if do_gather:
            pltpu.make_async_copy(eidx_hbm.at[w, m + 2], ibs[(q4 + 2) % 4],
                                  si2n).wait()
            pltpu.async_copy(g_hbm.at[ibs[(q4 + 2) % 4].at[0]], rb2n, sg2n)

    @pl.loop(0, _NLOOP, step=_UNROLL)
    def _edges(j):
        for t in range(_UNROLL):
            chunk_step(j + t, t % 3, t % 4, True, True)

    for m in range(_NLOOP, NCHUNK):
        chunk_step(m, m % 3, m % 4, m + 3 < NCHUNK, m + 2 < NCHUNK)

    plsc.subcore_barrier()
    pltpu.sync_copy(acc_sh.at[pl.ds(s * ROWS_SUB, ROWS_SUB), :],
                    out_hbm.at[c, pl.ds(s * ROWS_SUB, ROWS_SUB), :])


def _scatter_call(eidx_packed, g):
    k = pl.kernel(
        _scatter_body,
        out_type=jax.ShapeDtypeStruct((NUM_CORES, N_PAD, H), jnp.float32),
        mesh=_mesh(),
        scratch_types=[
            pltpu.VMEM((2, KS), jnp.int32),
            pltpu.VMEM((2, KS), jnp.int32),
            pltpu.VMEM((2, KS), jnp.int32),
            pltpu.VMEM((2, KS), jnp.int32),
            pltpu.VMEM((KS, H), jnp.float32),
            pltpu.VMEM((KS, H), jnp.float32),
            pltpu.VMEM((KS, H), jnp.float32),
            pltpu.VMEM_SHARED((N_PAD, H), jnp.float32),
            pltpu.SemaphoreType.DMA,
            pltpu.SemaphoreType.DMA,
            pltpu.SemaphoreType.DMA,
            pltpu.SemaphoreType.DMA,
            pltpu.SemaphoreType.DMA,
            pltpu.SemaphoreType.DMA,
            pltpu.SemaphoreType.DMA,
        ])
    return k(eidx_packed, g)


def _tc1_body(x_ref, w_ref, degp_ref, g_ref, dinv_ref):
    deg = degp_ref[:, 0] + degp_ref[:, 1] + 1.0
    dinv = lax.rsqrt(deg)[:, None]
    h = jnp.dot(x_ref[...], w_ref[...],
                preferred_element_type=jnp.float32,
                precision=lax.Precision.HIGHEST)
    g_ref[...] = h * dinv
    dinv_ref[...] = dinv


def _tc1_call(x, W1, degp):
    grid = N // TC_BLK
    degp_t = jnp.swapaxes(degp, 0, 1)
    return pl.pallas_call(
        _tc1_body,
        grid=(grid,),
        in_specs=[
            pl.BlockSpec((TC_BLK, D), lambda i: (i, 0)),
            pl.BlockSpec((D, H), lambda i: (0, 0)),
            pl.BlockSpec((TC_BLK, NUM_CORES), lambda i: (i, 0)),
        ],
        out_specs=[
            pl.BlockSpec((TC_BLK, H), lambda i: (i, 0)),
            pl.BlockSpec((TC_BLK, 1), lambda i: (i, 0)),
        ],
        out_shape=[
            jax.ShapeDtypeStruct((N, H), jnp.float32),
            jax.ShapeDtypeStruct((N, 1), jnp.float32),
        ],
    )(x, W1, degp_t)


def _tc2_body(p_ref, g_ref, dinv_ref, b_ref, w_ref, g2_ref):
    acc = p_ref[0] + p_ref[1] + g_ref[...]
    dinv = dinv_ref[...]
    t = jnp.tanh(acc * dinv + b_ref[...][None, :])
    h2 = jnp.dot(t, w_ref[...],
                 preferred_element_type=jnp.float32,
                 precision=lax.Precision.HIGHEST)
    g2_ref[...] = h2 * dinv


def _tc2_call(p1, g1, dinv, b1, W2):
    grid = N // TC_BLK
    return pl.pallas_call(
        _tc2_body,
        grid=(grid,),
        in_specs=[
            pl.BlockSpec((NUM_CORES, TC_BLK, H), lambda i: (0, i, 0)),
            pl.BlockSpec((TC_BLK, H), lambda i: (i, 0)),
            pl.BlockSpec((TC_BLK, 1), lambda i: (i, 0)),
            pl.BlockSpec((H,), lambda i: (0,)),
            pl.BlockSpec((H, H), lambda i: (0, 0)),
        ],
        out_specs=pl.BlockSpec((TC_BLK, H), lambda i: (i, 0)),
        out_shape=jax.ShapeDtypeStruct((N, H), jnp.float32),
    )(p1, g1, dinv, b1, W2)


def _tc3_body(p_ref, g_ref, dinv_ref, b_ref, wl_ref, bl_ref, o_ref):
    acc = p_ref[0] + p_ref[1] + g_ref[...]
    dinv = dinv_ref[...]
    t = jnp.tanh(acc * dinv + b_ref[...][None, :])
    o = jnp.dot(t, wl_ref[...],
                preferred_element_type=jnp.float32,
                precision=lax.Precision.HIGHEST) + bl_ref[...][None, :]
    m = jnp.max(o, axis=1, keepdims=True)
    e = jnp.exp(o - m)
    lse = jnp.log(jnp.sum(e, axis=1, keepdims=True))
    o_ref[...] = o - m - lse


def _tc3_call(p2, g2, dinv, b2, Wlin, blin):
    grid = N // TC_BLK
    return pl.pallas_call(
        _tc3_body,
        grid=(grid,),
        in_specs=[
            pl.BlockSpec((NUM_CORES, TC_BLK, H), lambda i: (0, i, 0)),
            pl.BlockSpec((TC_BLK, H), lambda i: (i, 0)),
            pl.BlockSpec((TC_BLK, 1), lambda i: (i, 0)),
            pl.BlockSpec((H,), lambda i: (0,)),
            pl.BlockSpec((H, C), lambda i: (0, 0)),
            pl.BlockSpec((C,), lambda i: (0,)),
        ],
        out_specs=pl.BlockSpec((TC_BLK, C), lambda i: (i, 0)),
        out_shape=jax.ShapeDtypeStruct((N, C), jnp.float32),
    )(p2, g2, dinv, b2, Wlin, blin)


def kernel(x, edge_index, W1, b1, W2, b2, Wlin, blin):
    dst = edge_index[1]
    eidx_packed = jnp.transpose(
        edge_index.reshape(2, NUM_W, NCHUNK, KS), (1, 2, 0, 3))
    degp = _deg_call(dst)
    g1, dinv = _tc1_call(x, W1, degp)
    p1 = _scatter_call(eidx_packed, g1)
    g2 = _tc2_call(p1, g1, dinv, b1, W2)
    p2 = _scatter_call(eidx_packed, g2)
    return _tc3_call(p2, g2, dinv, b2, Wlin, blin)
